# Initial kernel scaffold; baseline (speedup 1.0000x reference)
import jax
import jax.numpy as jnp
from jax.experimental import pallas as pl


def kernel(x, edge_index, norm_edge_weight, norm_self_loop, W1_0, b1_0, g1_0, be1_0, W2_0, b2_0, eps_0, go_0, bo_0, W1_1, b1_1, g1_1, be1_1, W2_1, b2_1, eps_1, go_1, bo_1):
    raise NotImplementedError("write your pallas kernel here")


# speedup vs baseline: 1.8391x; 1.8391x over previous
import functools
import math

import jax
import jax.numpy as jnp
from jax import lax
from jax.experimental import pallas as pl
from jax.experimental.pallas import tpu as pltpu
from jax.experimental.pallas import tpu_sc as plsc

N = 10000
E = 160000
EMB = 512
NC = 2
NS = 16
NSUB = 632
N_PAD = NS * NSUB
K = 80
PER_SUB = E // NS
NBATCH = PER_SUB // K

_BN_INV = 1.0 / math.sqrt(1.0 + 1e-5)


def _make_edge_agg(C, apply_relu):
  cpc = C // NC
  mesh = plsc.VectorSubcoreMesh(core_axis_name="c", subcore_axis_name="s",
                                num_cores=NC, num_subcores=NS)

  @functools.partial(
      pl.kernel,
      out_type=jax.ShapeDtypeStruct((C * N_PAD, 128), jnp.float32),
      mesh=mesh,
      scratch_types=[
          pltpu.VMEM((K,), jnp.int32),
          pltpu.VMEM((K,), jnp.int32),
          pltpu.VMEM((K, 16), jnp.float32),
          pltpu.VMEM((K,), jnp.int32),
          pltpu.VMEM((K, 128), jnp.float32),
          pltpu.VMEM((8, 128), jnp.float32),
          pltpu.VMEM_SHARED((N_PAD, 128), jnp.float32),
          pltpu.SemaphoreType.DMA,
      ],
      compiler_params=pltpu.CompilerParams(needs_layout_passes=False),
  )
  def edge_kernel(table, src_hbm, dst_hbm, wb_hbm, out,
                  src_v, dst_v, w16_v, gidx_v, rows_v, zer_v, agg_sh, sem):
    ci = lax.axis_index("c")
    si = lax.axis_index("s")
    for r in range(8):
      for j in range(8):
        zer_v[r, pl.ds(j * 16, 16)] = jnp.zeros((16,), jnp.float32)
    edge_base = si * PER_SUB

    for j in range(cpc):
      chunk = ci * cpc + j
      for t in range(NSUB // 8):
        pltpu.sync_copy(zer_v, agg_sh.at[pl.ds(si * NSUB + t * 8, 8)])
      plsc.subcore_barrier()

      row_off = chunk * N

      def batch_body(b, _):
        eb = pl.multiple_of(edge_base + b * K, 8)
        pltpu.sync_copy(src_hbm.at[pl.ds(eb, K)], src_v)
        pltpu.sync_copy(dst_hbm.at[pl.ds(eb, K)], dst_v)
        pltpu.sync_copy(wb_hbm.at[pl.ds(eb, K)], w16_v)
        for t in range(K // 16):
          gidx_v[pl.ds(t * 16, 16)] = src_v[pl.ds(t * 16, 16)] + row_off
        pltpu.async_copy(table.at[gidx_v], rows_v, sem).wait()
        for i in range(K):
          wi = w16_v[i, :]
          for q in range(8):
            r = rows_v[i, pl.ds(q * 16, 16)]
            if apply_relu:
              r = jnp.maximum(r, 0.0)
            rows_v[i, pl.ds(q * 16, 16)] = r * wi
        pltpu.sync_copy(rows_v, agg_sh.at[dst_v], add=True)
        return 0

      lax.fori_loop(0, NBATCH, batch_body, 0)
      plsc.subcore_barrier()
      dst_row = pl.multiple_of(chunk * N_PAD + si * NSUB, 8)
      pltpu.sync_copy(agg_sh.at[pl.ds(si * NSUB, NSUB)],
                      out.at[pl.ds(dst_row, NSUB)])
      plsc.subcore_barrier()

  return edge_kernel


R = 400
GRID = N // R


def _make_mlp(c_in, final):
  d_in = c_in * 128

  def body(xc_ref, agg_ref, nsl_ref, eps_ref, w1_ref, b1_ref, g1_ref, be1_ref,
           w2_ref, b2_ref, go_ref, bo_ref, out_ref):
    s = eps_ref[0, 0] * nsl_ref[...]
    parts = [s * xc_ref[c] + agg_ref[c] for c in range(c_in)]
    pre = jnp.concatenate(parts, axis=1)
    acc = jnp.dot(pre, w1_ref[...], preferred_element_type=jnp.float32, precision=lax.Precision.HIGHEST)
    acc = acc + b1_ref[...]
    t = jnp.maximum(acc * (_BN_INV * g1_ref[...]) + be1_ref[...], 0.0)
    u = jnp.dot(t, w2_ref[...], preferred_element_type=jnp.float32, precision=lax.Precision.HIGHEST)
    u = u + b2_ref[...]
    h = jnp.maximum(u * (_BN_INV * go_ref[...]) + bo_ref[...], 0.0)
    if final:
      xcat = jnp.concatenate([xc_ref[c] for c in range(c_in)], axis=1)
      out_ref[...] = xcat + 2.0 * h
    else:
      for c in range(4):
        out_ref[c] = h[:, c * 128:(c + 1) * 128]

  whole = lambda i: (0, 0)
  in_specs = [
      pl.BlockSpec((c_in, R, 128), lambda i: (0, i, 0)),
      pl.BlockSpec((c_in, R, 128), lambda i: (0, i, 0)),
      pl.BlockSpec((R, 1), lambda i: (i, 0)),
      pl.BlockSpec((1, 1), whole),
      pl.BlockSpec((d_in, EMB), whole),
      pl.BlockSpec((1, EMB), whole),
      pl.BlockSpec((1, EMB), whole),
      pl.BlockSpec((1, EMB), whole),
      pl.BlockSpec((EMB, EMB), whole),
      pl.BlockSpec((1, EMB), whole),
      pl.BlockSpec((1, EMB), whole),
      pl.BlockSpec((1, EMB), whole),
  ]
  if final:
    out_spec = pl.BlockSpec((R, EMB), lambda i: (i, 0))
    out_shape = jax.ShapeDtypeStruct((N, EMB), jnp.float32)
  else:
    out_spec = pl.BlockSpec((4, R, 128), lambda i: (0, i, 0))
    out_shape = jax.ShapeDtypeStruct((4, N, 128), jnp.float32)

  return pl.pallas_call(
      body,
      grid=(GRID,),
      in_specs=in_specs,
      out_specs=out_spec,
      out_shape=out_shape,
  )


_make_edge_agg = functools.lru_cache(None)(_make_edge_agg)
_make_mlp = functools.lru_cache(None)(_make_mlp)


def _edge0(*a):
  return _make_edge_agg(2, apply_relu=True)(*a)


def _edge1(*a):
  return _make_edge_agg(4, apply_relu=False)(*a)


def _mlp0(*a):
  return _make_mlp(2, final=False)(*a)


def _mlp1(*a):
  return _make_mlp(4, final=True)(*a)


def kernel(x, edge_index, norm_edge_weight, norm_self_loop,
           W1_0, b1_0, g1_0, be1_0, W2_0, b2_0, eps_0, go_0, bo_0,
           W1_1, b1_1, g1_1, be1_1, W2_1, b2_1, eps_1, go_1, bo_1):
  src = edge_index[0]
  dst = edge_index[1]
  wb = jnp.repeat(norm_edge_weight[:, None], 16, axis=1)
  nsl = norm_self_loop.reshape(N, 1)

  def row(v):
    return v.reshape(1, EMB)

  xc = jnp.transpose(x.reshape(N, 2, 128), (1, 0, 2))
  agg0 = _edge0(xc.reshape(2 * N, 128), src, dst, wb)
  agg0 = agg0.reshape(2, N_PAD, 128)
  h1c = _mlp0(xc, agg0, nsl, eps_0.reshape(1, 1),
              W1_0, row(b1_0), row(g1_0), row(be1_0),
              W2_0, row(b2_0), row(go_0), row(bo_0))
  agg1 = _edge1(h1c.reshape(4 * N, 128), src, dst, wb)
  agg1 = agg1.reshape(4, N_PAD, 128)
  out = _mlp1(h1c, agg1, nsl, eps_1.reshape(1, 1),
              W1_1, row(b1_1), row(g1_1), row(be1_1),
              W2_1, row(b2_1), row(go_1), row(bo_1))
  return out


# speedup vs baseline: 2.4752x; 1.3458x over previous
import functools
import math

import jax
import jax.numpy as jnp
from jax import lax
from jax.experimental import pallas as pl
from jax.experimental.pallas import tpu as pltpu
from jax.experimental.pallas import tpu_sc as plsc

N = 10000
E = 160000
EMB = 512
NC = 2
NS = 16
NSUB = 632
N_PAD = NS * NSUB
K = 80
PER_SUB = 10080
NBATCH = PER_SUB // K
E_PAD = NS * PER_SUB

_BN_INV = 1.0 / math.sqrt(1.0 + 1e-5)


def _make_edge_agg(C, apply_relu):
  cpc = C // NC
  mesh = plsc.VectorSubcoreMesh(core_axis_name="c", subcore_axis_name="s",
                                num_cores=NC, num_subcores=NS)

  @functools.partial(
      pl.kernel,
      out_type=jax.ShapeDtypeStruct((C * N_PAD, 128), jnp.float32),
      mesh=mesh,
      scratch_types=[
          [pltpu.VMEM((K,), jnp.int32)] * 2,
          [pltpu.VMEM((K,), jnp.int32)] * 2,
          [pltpu.VMEM((K, 16), jnp.float32)] * 2,
          [pltpu.VMEM((K,), jnp.int32)] * 2,
          [pltpu.VMEM((K, 128), jnp.float32)] * 2,
          pltpu.VMEM((8, 128), jnp.float32),
          pltpu.VMEM_SHARED((N_PAD, 128), jnp.float32),
          [pltpu.SemaphoreType.DMA] * 2,
          [pltpu.SemaphoreType.DMA] * 2,
      ],
      compiler_params=pltpu.CompilerParams(needs_layout_passes=False),
  )
  def edge_kernel(table, src_hbm, dst_hbm, wb_hbm, out,
                  src_v, dst_v, w16_v, gidx_v, rows_v, zer_v, agg_sh,
                  sem_m, sem_g):
    ci = lax.axis_index("c")
    si = lax.axis_index("s")
    for r in range(8):
      for j in range(8):
        zer_v[r, pl.ds(j * 16, 16)] = jnp.zeros((16,), jnp.float32)
    edge_base = si * PER_SUB

    def meta_start(b, s):
      eb = pl.multiple_of(edge_base + b * K, 8)
      pltpu.async_copy(src_hbm.at[pl.ds(eb, K)], src_v[s], sem_m[s])
      pltpu.async_copy(dst_hbm.at[pl.ds(eb, K)], dst_v[s], sem_m[s])
      pltpu.async_copy(wb_hbm.at[pl.ds(eb, K)], w16_v[s], sem_m[s])

    def meta_wait(s):
      pltpu.make_async_copy(src_hbm.at[pl.ds(0, K)], src_v[s], sem_m[s]).wait()
      pltpu.make_async_copy(dst_hbm.at[pl.ds(0, K)], dst_v[s], sem_m[s]).wait()
      pltpu.make_async_copy(wb_hbm.at[pl.ds(0, K)], w16_v[s], sem_m[s]).wait()

    def gather_start(s, row_off):
      for t in range(K // 16):
        gidx_v[s][pl.ds(t * 16, 16)] = src_v[s][pl.ds(t * 16, 16)] + row_off
      pltpu.async_copy(table.at[gidx_v[s]], rows_v[s], sem_g[s])

    def gather_wait(s):
      pltpu.make_async_copy(table.at[gidx_v[s]], rows_v[s], sem_g[s]).wait()

    def scale_scatter(s):
      for i in range(K):
        wi = w16_v[s][i, :]
        for q in range(8):
          r = rows_v[s][i, pl.ds(q * 16, 16)]
          if apply_relu:
            r = jnp.maximum(r, 0.0)
          rows_v[s][i, pl.ds(q * 16, 16)] = r * wi
      pltpu.sync_copy(rows_v[s], agg_sh.at[dst_v[s]], add=True)

    for j in range(cpc):
      chunk = ci * cpc + j
      for t in range(NSUB // 8):
        pltpu.sync_copy(zer_v, agg_sh.at[pl.ds(si * NSUB + t * 8, 8)])
      plsc.subcore_barrier()

      row_off = chunk * N

      meta_start(0, 0)
      meta_wait(0)
      gather_start(0, row_off)
      meta_start(1, 1)

      def pair_body(b2, _):
        b = b2 * 2
        meta_wait(1)
        gather_start(1, row_off)
        gather_wait(0)
        scale_scatter(0)
        meta_start(b + 2, 0)
        meta_wait(0)
        gather_start(0, row_off)
        gather_wait(1)
        scale_scatter(1)
        meta_start(b + 3, 1)
        return 0

      lax.fori_loop(0, NBATCH // 2 - 1, pair_body, 0)
      meta_wait(1)
      gather_start(1, row_off)
      gather_wait(0)
      scale_scatter(0)
      gather_wait(1)
      scale_scatter(1)

      plsc.subcore_barrier()
      dst_row = pl.multiple_of(chunk * N_PAD + si * NSUB, 8)
      pltpu.sync_copy(agg_sh.at[pl.ds(si * NSUB, NSUB)],
                      out.at[pl.ds(dst_row, NSUB)])
      plsc.subcore_barrier()

  return edge_kernel


R = 400
GRID = N // R


def _make_mlp(c_in, final):
  d_in = c_in * 128

  def body(xc_ref, agg_ref, nsl_ref, eps_ref, w1_ref, b1_ref, g1_ref, be1_ref,
           w2_ref, b2_ref, go_ref, bo_ref, out_ref):
    s = eps_ref[0, 0] * nsl_ref[...]
    parts = [s * xc_ref[c] + agg_ref[c] for c in range(c_in)]
    pre = jnp.concatenate(parts, axis=1)
    acc = jnp.dot(pre, w1_ref[...], preferred_element_type=jnp.float32, precision=lax.Precision.HIGHEST)
    acc = acc + b1_ref[...]
    t = jnp.maximum(acc * (_BN_INV * g1_ref[...]) + be1_ref[...], 0.0)
    u = jnp.dot(t, w2_ref[...], preferred_element_type=jnp.float32, precision=lax.Precision.HIGHEST)
    u = u + b2_ref[...]
    h = jnp.maximum(u * (_BN_INV * go_ref[...]) + bo_ref[...], 0.0)
    if final:
      xcat = jnp.concatenate([xc_ref[c] for c in range(c_in)], axis=1)
      out_ref[...] = xcat + 2.0 * h
    else:
      for c in range(4):
        out_ref[c] = h[:, c * 128:(c + 1) * 128]

  whole = lambda i: (0, 0)
  in_specs = [
      pl.BlockSpec((c_in, R, 128), lambda i: (0, i, 0)),
      pl.BlockSpec((c_in, R, 128), lambda i: (0, i, 0)),
      pl.BlockSpec((R, 1), lambda i: (i, 0)),
      pl.BlockSpec((1, 1), whole),
      pl.BlockSpec((d_in, EMB), whole),
      pl.BlockSpec((1, EMB), whole),
      pl.BlockSpec((1, EMB), whole),
      pl.BlockSpec((1, EMB), whole),
      pl.BlockSpec((EMB, EMB), whole),
      pl.BlockSpec((1, EMB), whole),
      pl.BlockSpec((1, EMB), whole),
      pl.BlockSpec((1, EMB), whole),
  ]
  if final:
    out_spec = pl.BlockSpec((R, EMB), lambda i: (i, 0))
    out_shape = jax.ShapeDtypeStruct((N, EMB), jnp.float32)
  else:
    out_spec = pl.BlockSpec((4, R, 128), lambda i: (0, i, 0))
    out_shape = jax.ShapeDtypeStruct((4, N, 128), jnp.float32)

  return pl.pallas_call(
      body,
      grid=(GRID,),
      in_specs=in_specs,
      out_specs=out_spec,
      out_shape=out_shape,
  )


_make_edge_agg = functools.lru_cache(None)(_make_edge_agg)
_make_mlp = functools.lru_cache(None)(_make_mlp)


def _edge0(*a):
  return _make_edge_agg(2, apply_relu=True)(*a)


def _edge1(*a):
  return _make_edge_agg(4, apply_relu=False)(*a)


def _mlp0(*a):
  return _make_mlp(2, final=False)(*a)


def _mlp1(*a):
  return _make_mlp(4, final=True)(*a)


def kernel(x, edge_index, norm_edge_weight, norm_self_loop,
           W1_0, b1_0, g1_0, be1_0, W2_0, b2_0, eps_0, go_0, bo_0,
           W1_1, b1_1, g1_1, be1_1, W2_1, b2_1, eps_1, go_1, bo_1):
  def pad_edges(v):
    return jnp.pad(v.reshape(NS, E // NS), ((0, 0), (0, PER_SUB - E // NS)))

  src = pad_edges(edge_index[0]).reshape(E_PAD)
  dst = pad_edges(edge_index[1]).reshape(E_PAD)
  wb = jnp.repeat(pad_edges(norm_edge_weight).reshape(E_PAD, 1), 16, axis=1)
  nsl = norm_self_loop.reshape(N, 1)

  def row(v):
    return v.reshape(1, EMB)

  xc = jnp.transpose(x.reshape(N, 2, 128), (1, 0, 2))
  agg0 = _edge0(xc.reshape(2 * N, 128), src, dst, wb)
  agg0 = agg0.reshape(2, N_PAD, 128)
  h1c = _mlp0(xc, agg0, nsl, eps_0.reshape(1, 1),
              W1_0, row(b1_0), row(g1_0), row(be1_0),
              W2_0, row(b2_0), row(go_0), row(bo_0))
  agg1 = _edge1(h1c.reshape(4 * N, 128), src, dst, wb)
  agg1 = agg1.reshape(4, N_PAD, 128)
  out = _mlp1(h1c, agg1, nsl, eps_1.reshape(1, 1),
              W1_1, row(b1_1), row(g1_1), row(be1_1),
              W2_1, row(b2_1), row(go_1), row(bo_1))
  return out
